# Initial kernel scaffold; baseline (speedup 1.0000x reference)
import jax
import jax.numpy as jnp
from jax.experimental import pallas as pl


def kernel(x):
    raise NotImplementedError("write your pallas kernel here")


# speedup vs baseline: 1.2560x; 1.2560x over previous
import jax
import jax.numpy as jnp
from jax import lax
from jax.experimental import pallas as pl
from jax.experimental.pallas import tpu as pltpu
from jax.experimental.pallas import tpu_sc as plsc

N_ROWS = 16384
N_FIELDS = 26
FIELD_N = 100
ROW_W = N_FIELDS * FIELD_N
NC, NS = 2, 16
NW = NC * NS
ROWS_PER_W = N_ROWS // NW
T = 16
CHUNKS = ROWS_PER_W // T
BUF_W = T * ROW_W
XW = ROWS_PER_W * N_FIELDS


def _body(x_hbm, out_hbm, xbuf, buf, sem0, sem1):
    wid = lax.axis_index("c") * NS + lax.axis_index("s")
    lanes = lax.iota(jnp.int32, 16)
    ones = jnp.full((16,), 1, jnp.int32)
    zeros = jnp.zeros((16,), jnp.int32)
    sems = (sem0, sem1)
    out_base = wid * ROWS_PER_W * ROW_W

    pltpu.sync_copy(x_hbm.at[pl.ds(wid * XW, XW)], xbuf)

    def zero_body(k, carry):
        buf[pl.ds(k * 16, 16)] = zeros
        return carry

    lax.fori_loop(0, (2 * BUF_W) // 16, zero_body, 0)

    def scatter(c, b, val):
        for i in range(N_FIELDS):
            v = plsc.load_gather(xbuf, [(c * T + lanes) * N_FIELDS + i])
            plsc.store_scatter(
                buf, [b * BUF_W + lanes * ROW_W + (i * FIELD_N) + v], val)

    def start_stream(c, b):
        pltpu.async_copy(
            buf.at[pl.ds(b * BUF_W, BUF_W)],
            out_hbm.at[pl.ds(out_base + c * BUF_W, BUF_W)],
            sems[b])

    def wait_stream(b):
        pltpu.make_async_copy(
            buf.at[pl.ds(b * BUF_W, BUF_W)],
            out_hbm.at[pl.ds(out_base, BUF_W)],
            sems[b]).wait()

    for b in range(2):
        scatter(jnp.int32(b), b, ones)
        start_stream(jnp.int32(b), b)

    def loop_body(it, carry):
        g = it * 2
        for b in range(2):
            c = g + b
            wait_stream(b)
            scatter(c - 2, b, zeros)
            scatter(c, b, ones)
            start_stream(c, b)
        return carry

    lax.fori_loop(1, CHUNKS // 2, loop_body, 0)

    for b in range(2):
        wait_stream(b)


@jax.jit
def _run(xf):
    mesh = plsc.VectorSubcoreMesh(core_axis_name="c", subcore_axis_name="s")
    f = pl.kernel(
        _body,
        out_type=jax.ShapeDtypeStruct((N_ROWS * ROW_W,), jnp.int32),
        mesh=mesh,
        scratch_types=[
            pltpu.VMEM((XW,), jnp.int32),
            pltpu.VMEM((2 * BUF_W,), jnp.int32),
            pltpu.SemaphoreType.DMA,
            pltpu.SemaphoreType.DMA,
        ],
        compiler_params=pltpu.CompilerParams(needs_layout_passes=False),
    )
    return f(xf)


def kernel(x):
    out = _run(x.reshape(-1))
    return out.reshape(N_ROWS, ROW_W)


# speedup vs baseline: 5.2180x; 4.1546x over previous
import jax
import jax.numpy as jnp
from jax import lax
from jax.experimental import pallas as pl
from jax.experimental.pallas import tpu as pltpu
from jax.experimental.pallas import tpu_sc as plsc

N_ROWS = 16384
N_FIELDS = 26
FIELD_N = 100
ROW_W = N_FIELDS * FIELD_N
NC, NS = 2, 16
NW = NC * NS
ROWS_PER_W = N_ROWS // NW
BC = 200
BR = 128
SGS = ROW_W // BC
RTS = ROWS_PER_W // BR
CHUNKS = SGS * RTS
XW = ROWS_PER_W * N_FIELDS


def _body(x_hbm, out_hbm, xbuf, buf, sem0, sem1):
    wid = lax.axis_index("c") * NS + lax.axis_index("s")
    lanes = lax.iota(jnp.int32, 16)
    ones = jnp.full((16,), 1, jnp.int32)
    zeros = jnp.zeros((16,), jnp.int32)
    sems = (sem0, sem1)
    row0 = wid * ROWS_PER_W

    pltpu.sync_copy(x_hbm.at[pl.ds(wid * XW, XW)], xbuf)

    for b in range(2):
        def zero_body(k, carry, b=b):
            buf[b, k >> 3, pl.ds((k & 7) * 16, 16)] = zeros
            return carry

        lax.fori_loop(0, BC * BR // 16, zero_body, 0)

    def scatter(c, b, val):
        sg = c >> 2
        rt = c & 3
        bsplat = jnp.full((16,), b, jnp.int32)
        for ii in range(2):
            for g in range(8):
                rloc = rt * BR + g * 16 + lanes
                xv = plsc.load_gather(xbuf, [rloc * N_FIELDS + 2 * sg + ii])
                plsc.store_scatter(
                    buf, [bsplat, ii * FIELD_N + xv, g * 16 + lanes], val)

    def start_stream(c, b):
        sg = c >> 2
        rt = c & 3
        pltpu.async_copy(
            buf.at[b],
            out_hbm.at[pl.ds(sg * BC, BC), pl.ds(row0 + rt * BR, BR)],
            sems[b])

    def wait_stream(b):
        pltpu.make_async_copy(
            buf.at[b],
            out_hbm.at[pl.ds(0, BC), pl.ds(0, BR)],
            sems[b]).wait()

    for b in range(2):
        scatter(jnp.int32(b), b, ones)
        start_stream(jnp.int32(b), b)

    def loop_body(it, carry):
        for b in range(2):
            c = it * 2 + b
            wait_stream(b)
            scatter(c - 2, b, zeros)
            scatter(c, b, ones)
            start_stream(c, b)
        return carry

    lax.fori_loop(1, CHUNKS // 2, loop_body, 0)

    for b in range(2):
        wait_stream(b)


@jax.jit
def _run(xf):
    mesh = plsc.VectorSubcoreMesh(core_axis_name="c", subcore_axis_name="s")
    f = pl.kernel(
        _body,
        out_type=jax.ShapeDtypeStruct((ROW_W, N_ROWS), jnp.int32),
        mesh=mesh,
        scratch_types=[
            pltpu.VMEM((XW,), jnp.int32),
            pltpu.VMEM((2, BC, BR), jnp.int32),
            pltpu.SemaphoreType.DMA,
            pltpu.SemaphoreType.DMA,
        ],
        compiler_params=pltpu.CompilerParams(needs_layout_passes=False),
    )
    return f(xf)


def kernel(x):
    return jnp.transpose(_run(x.reshape(-1)))


# speedup vs baseline: 5.9395x; 1.1383x over previous
import jax
import jax.numpy as jnp
from jax import lax
from jax.experimental import pallas as pl
from jax.experimental.pallas import tpu as pltpu
from jax.experimental.pallas import tpu_sc as plsc

N_ROWS = 16384
N_FIELDS = 26
FIELD_N = 100
ROW_W = N_FIELDS * FIELD_N
NC, NS = 2, 16
NW = NC * NS
ROWS_PER_W = N_ROWS // NW
BC = 200
BR = 128
SGS = ROW_W // BC
RTS = ROWS_PER_W // BR
CHUNKS = SGS * RTS
XW = ROWS_PER_W * N_FIELDS


def _body(x_hbm, out_hbm, xbuf, buf, sem0, sem1):
    wid = lax.axis_index("c") * NS + lax.axis_index("s")
    lanes = lax.iota(jnp.int32, 16)
    ones = jnp.full((16,), 1, jnp.int32)
    zeros = jnp.zeros((16,), jnp.int32)
    sems = (sem0, sem1)
    row0 = wid * ROWS_PER_W

    pltpu.sync_copy(x_hbm.at[:, pl.ds(row0, ROWS_PER_W)], xbuf)

    for b in range(2):
        def zero_body(k, carry, b=b):
            buf[b, k >> 3, pl.ds((k & 7) * 16, 16)] = zeros
            return carry

        lax.fori_loop(0, BC * BR // 16, zero_body, 0)

    def scatter(c, b, val):
        sg = c >> 2
        rt = c & 3
        bsplat = jnp.full((16,), b, jnp.int32)
        for ii in range(2):
            fvec = jnp.full((16,), 2 * sg + ii, jnp.int32)
            for g in range(8):
                rloc = rt * BR + g * 16 + lanes
                xv = plsc.load_gather(xbuf, [fvec, rloc])
                plsc.store_scatter(
                    buf, [bsplat, ii * FIELD_N + xv, g * 16 + lanes], val)

    def start_stream(c, b):
        sg = c >> 2
        rt = c & 3
        pltpu.async_copy(
            buf.at[b],
            out_hbm.at[pl.ds(sg * BC, BC), pl.ds(row0 + rt * BR, BR)],
            sems[b])

    def wait_stream(b):
        pltpu.make_async_copy(
            buf.at[b],
            out_hbm.at[pl.ds(0, BC), pl.ds(0, BR)],
            sems[b]).wait()

    for b in range(2):
        scatter(jnp.int32(b), b, ones)
        start_stream(jnp.int32(b), b)

    def loop_body(it, carry):
        for b in range(2):
            c = it * 2 + b
            wait_stream(b)
            scatter(c - 2, b, zeros)
            scatter(c, b, ones)
            start_stream(c, b)
        return carry

    lax.fori_loop(1, CHUNKS // 2, loop_body, 0)

    for b in range(2):
        wait_stream(b)


@jax.jit
def _run(xt):
    mesh = plsc.VectorSubcoreMesh(core_axis_name="c", subcore_axis_name="s")
    f = pl.kernel(
        _body,
        out_type=jax.ShapeDtypeStruct((ROW_W, N_ROWS), jnp.int32),
        mesh=mesh,
        scratch_types=[
            pltpu.VMEM((N_FIELDS, ROWS_PER_W), jnp.int32),
            pltpu.VMEM((2, BC, BR), jnp.int32),
            pltpu.SemaphoreType.DMA,
            pltpu.SemaphoreType.DMA,
        ],
        compiler_params=pltpu.CompilerParams(needs_layout_passes=False),
    )
    return f(xt)


def kernel(x):
    return jnp.transpose(_run(jnp.transpose(x)))
